# baseline (speedup 1.0000x reference)
import functools

import jax
import jax.numpy as jnp
from jax import lax
from jax.experimental import pallas as pl

_BETA = 0.5
_MSB = -0x80000000


def _float_keys(hm):
    u = lax.bitcast_convert_type(hm, jnp.int32)
    signed = jnp.where(u >= 0, u, u ^ jnp.int32(0x7FFFFFFF))
    return signed ^ jnp.int32(_MSB)


def _cnt(pred):
    return jnp.where(pred, jnp.int32(1), jnp.int32(0))


def _kth_largest(fkeys, k):
    pmask = jnp.int32(0)
    pval = jnp.int32(0)
    kk = jnp.int32(k)
    for s in range(16):
        sh = 30 - 2 * s
        q = (fkeys >> sh) & jnp.int32(3)
        matches = (fkeys & pmask) == pval
        r1 = jnp.sum(_cnt(matches & (q == 3))
                     + (_cnt(matches & (q == 2)) << 11))
        c1 = jnp.sum(_cnt(matches & (q == 1)))
        c3 = r1 & jnp.int32(0x7FF)
        c2 = r1 >> 11
        t3 = c3
        t2 = c3 + c2
        t1 = t2 + c1
        sel3 = kk <= t3
        sel2 = (~sel3) & (kk <= t2)
        sel1 = (~sel3) & (~sel2) & (kk <= t1)
        pick = jnp.where(
            sel3, jnp.int32(3),
            jnp.where(sel2, jnp.int32(2),
                      jnp.where(sel1, jnp.int32(1), jnp.int32(0))))
        sub = jnp.where(
            sel3, jnp.int32(0),
            jnp.where(sel2, t3, jnp.where(sel1, t2, t1)))
        pmask = pmask | (jnp.int32(3) << sh)
        pval = pval | (pick << sh)
        kk = kk - sub
    return pval, kk


def _tie_index_bound(eq, iota, r):
    base = jnp.int32(0)
    for s in range(5):
        w = jnp.int32(256 >> (2 * s))
        r1 = jnp.sum(_cnt(eq & (iota <= base + w - 1))
                     + (_cnt(eq & (iota <= base + 2 * w - 1)) << 11))
        cc = jnp.sum(_cnt(eq & (iota <= base + 3 * w - 1)))
        ca = r1 & jnp.int32(0x7FF)
        cb = r1 >> 11
        step = jnp.where(
            ca >= r, jnp.int32(0),
            jnp.where(cb >= r, w, jnp.where(cc >= r, 2 * w, 3 * w)))
        base = base + step
    return base


def _compute_mask(hm, k):
    fkeys = _float_keys(hm)
    t, r = _kth_largest(fkeys, k)
    keys = fkeys ^ jnp.int32(_MSB)
    tt = t ^ jnp.int32(_MSB)
    iota = (lax.broadcasted_iota(jnp.int32, hm.shape, 0) * 128
            + lax.broadcasted_iota(jnp.int32, hm.shape, 1))
    eq = keys == tt
    j = _tie_index_bound(eq, iota, r)
    drop = (keys > tt) | (eq & (iota <= j))
    return jnp.where(drop, jnp.float32(0.0), jnp.float32(1.0))


def _heblock_body(x_ref, o_ref, *, k):
    C = x_ref.shape[0]
    accs = [x_ref[i] for i in range(8)]
    for c in range(8, C, 8):
        for i in range(8):
            accs[i] = accs[i] + x_ref[c + i]
    hm = ((accs[0] + accs[1]) + (accs[2] + accs[3])) + (
        (accs[4] + accs[5]) + (accs[6] + accs[7]))
    mask = _compute_mask(hm, k)
    o_ref[...] = x_ref[...] * mask[None, :, :]


def kernel(x):
    B, C, H, W = x.shape
    n = H * W
    k = int(_BETA * n)
    x2 = x.reshape(B, C, n // 128, 128)
    body = functools.partial(_heblock_body, k=k)
    out = pl.pallas_call(
        body,
        grid=(B,),
        in_specs=[pl.BlockSpec((None, C, n // 128, 128), lambda b: (b, 0, 0, 0))],
        out_specs=pl.BlockSpec((None, C, n // 128, 128), lambda b: (b, 0, 0, 0)),
        out_shape=jax.ShapeDtypeStruct((B, C, n // 128, 128), jnp.float32),
    )(x2)
    return out.reshape(B, C, H, W)


# speedup vs baseline: 1.0585x; 1.0585x over previous
import functools

import jax
import jax.numpy as jnp
from jax import lax
from jax.experimental import pallas as pl

_BETA = 0.5
_MSB = -0x80000000


def _float_keys(hm):
    u = lax.bitcast_convert_type(hm, jnp.int32)
    signed = jnp.where(u >= 0, u, u ^ jnp.int32(0x7FFFFFFF))
    return signed ^ jnp.int32(_MSB)


def _cnt(pred):
    return jnp.where(pred, jnp.int32(1), jnp.int32(0))


def _kth_largest(fkeys, k):
    pmask = jnp.int32(0)
    pval = jnp.int32(0)
    kk = jnp.int32(k)
    for s in range(8):
        sh = 28 - 4 * s
        nib = (fkeys >> sh) & jnp.int32(15)
        matches = (fkeys & pmask) == pval
        packed = [
            jnp.sum(_cnt(matches & (nib == hi))
                    + (_cnt(matches & (nib == hi - 1)) << 11))
            for hi in (15, 13, 11, 9, 7, 5, 3)
        ]
        c1 = jnp.sum(_cnt(matches & (nib == 1)))
        c = [jnp.int32(0)] * 16
        for j, hi in enumerate((15, 13, 11, 9, 7, 5, 3)):
            c[hi] = packed[j] & jnp.int32(0x7FF)
            c[hi - 1] = packed[j] >> 11
        c[1] = c1
        t = [jnp.int32(0)] * 17
        acc = jnp.int32(0)
        for v in range(15, 0, -1):
            acc = acc + c[v]
            t[v] = acc
        pick = jnp.int32(0)
        sub = t[1]
        for v in range(1, 16):
            hit = kk <= t[v]
            pick = jnp.where(hit, jnp.int32(v), pick)
            sub = jnp.where(hit, t[v + 1] if v < 15 else jnp.int32(0), sub)
        pmask = pmask | (jnp.int32(15) << sh)
        pval = pval | (pick << sh)
        kk = kk - sub
    return pval, kk


def _tie_index_bound(eq, iota, r):
    base = jnp.int32(0)
    for s in range(5):
        w = jnp.int32(256 >> (2 * s))
        r1 = jnp.sum(_cnt(eq & (iota <= base + w - 1))
                     + (_cnt(eq & (iota <= base + 2 * w - 1)) << 11))
        cc = jnp.sum(_cnt(eq & (iota <= base + 3 * w - 1)))
        ca = r1 & jnp.int32(0x7FF)
        cb = r1 >> 11
        step = jnp.where(
            ca >= r, jnp.int32(0),
            jnp.where(cb >= r, w, jnp.where(cc >= r, 2 * w, 3 * w)))
        base = base + step
    return base


def _compute_mask(hm, k):
    fkeys = _float_keys(hm)
    t, r = _kth_largest(fkeys, k)
    keys = fkeys ^ jnp.int32(_MSB)
    tt = t ^ jnp.int32(_MSB)
    iota = (lax.broadcasted_iota(jnp.int32, hm.shape, 0) * 128
            + lax.broadcasted_iota(jnp.int32, hm.shape, 1))
    eq = keys == tt
    j = _tie_index_bound(eq, iota, r)
    drop = (keys > tt) | (eq & (iota <= j))
    return jnp.where(drop, jnp.float32(0.0), jnp.float32(1.0))


def _heblock_body(x_ref, o_ref, *, k):
    C = x_ref.shape[0]
    accs = [x_ref[i] for i in range(8)]
    for c in range(8, C, 8):
        for i in range(8):
            accs[i] = accs[i] + x_ref[c + i]
    hm = ((accs[0] + accs[1]) + (accs[2] + accs[3])) + (
        (accs[4] + accs[5]) + (accs[6] + accs[7]))
    mask = _compute_mask(hm, k)
    o_ref[...] = x_ref[...] * mask[None, :, :]


def kernel(x):
    B, C, H, W = x.shape
    n = H * W
    k = int(_BETA * n)
    x2 = x.reshape(B, C, n // 128, 128)
    body = functools.partial(_heblock_body, k=k)
    out = pl.pallas_call(
        body,
        grid=(B,),
        in_specs=[pl.BlockSpec((None, C, n // 128, 128), lambda b: (b, 0, 0, 0))],
        out_specs=pl.BlockSpec((None, C, n // 128, 128), lambda b: (b, 0, 0, 0)),
        out_shape=jax.ShapeDtypeStruct((B, C, n // 128, 128), jnp.float32),
    )(x2)
    return out.reshape(B, C, H, W)
